# baseline (speedup 1.0000x reference)
import functools

import jax
import jax.numpy as jnp
from jax.experimental import pallas as pl
from jax.experimental.pallas import tpu as pltpu

_PAR2 = pltpu.CompilerParams(dimension_semantics=("parallel", "parallel"))

_K = 32


def _split_limbs(a):
    a_hi = a.astype(jnp.bfloat16).astype(jnp.float32)
    return a_hi, a - a_hi


def _onehot_gather(onehot, a_hi, a_lo):
    o = onehot.astype(jnp.float32)
    return (jnp.dot(o, a_hi, preferred_element_type=jnp.float32)
            + jnp.dot(o, a_lo, preferred_element_type=jnp.float32))


def _fold_bn(layer):
    scale = layer['gamma'] / jnp.sqrt(layer['var'] + 1e-3)
    w = layer['W'] * scale[None, :]
    b = layer['beta'] - layer['mean'] * scale
    return w, b.reshape(1, -1)


def _fps_kernel(xt_ref, yt_ref, zt_ref, ox_ref, oy_ref, oz_ref, *, npoint, n):
    xs = xt_ref[...]
    ys = yt_ref[...]
    zs = zt_ref[...]
    bsz = xs.shape[0]
    iota = jax.lax.broadcasted_iota(jnp.int32, (bsz, n), 1)
    oiota = jax.lax.broadcasted_iota(jnp.int32, (bsz, npoint), 1)

    def body(i, carry):
        distance, farthest, ax, ay, az = carry
        onehot = iota == farthest
        cx = jnp.sum(jnp.where(onehot, xs, 0.0), axis=1, keepdims=True)
        cy = jnp.sum(jnp.where(onehot, ys, 0.0), axis=1, keepdims=True)
        cz = jnp.sum(jnp.where(onehot, zs, 0.0), axis=1, keepdims=True)
        osel = oiota == i
        ax = ax + jnp.where(osel, cx, 0.0)
        ay = ay + jnp.where(osel, cy, 0.0)
        az = az + jnp.where(osel, cz, 0.0)
        dx = xs - cx
        dy = ys - cy
        dz = zs - cz
        d = dx * dx + dy * dy
        d = d + dz * dz
        distance = jnp.minimum(distance, d)
        m = jnp.max(distance, axis=1, keepdims=True)
        farthest = jnp.min(jnp.where(distance == m, iota, n), axis=1,
                           keepdims=True)
        return distance, farthest, ax, ay, az

    dist0 = jnp.full((bsz, n), 1e10, jnp.float32)
    far0 = jnp.zeros((bsz, 1), jnp.int32)
    acc0 = jnp.zeros((bsz, npoint), jnp.float32)
    _, _, ax, ay, az = jax.lax.fori_loop(
        0, npoint, body, (dist0, far0, acc0, acc0, acc0))
    ox_ref[...] = ax
    oy_ref[...] = ay
    oz_ref[...] = az


def _fps(xt, yt, zt, npoint):
    bsz, n = xt.shape
    shp = jax.ShapeDtypeStruct((bsz, npoint), jnp.float32)
    ox, oy, oz = pl.pallas_call(
        functools.partial(_fps_kernel, npoint=npoint, n=n),
        out_shape=(shp, shp, shp),
    )(xt, yt, zt)
    return jnp.stack([ox, oy, oz], axis=-1), ox, oy, oz


def _sa_kernel(*refs, has_feat, n, s_blk):
    nx_ref, xt_ref, yt_ref, zt_ref, xyz_ref = refs[:5]
    p = 5
    feat_ref = None
    if has_feat:
        feat_ref = refs[p]
        p += 1
    w1_ref, b1_ref, w2_ref, b2_ref, w3_ref, b3_ref = refs[p:p + 6]
    out_ref = refs[p + 6]

    nx = nx_ref[0]
    d = nx[:, 0:1] - xt_ref[0]
    dist = d * d
    d = nx[:, 1:2] - yt_ref[0]
    dist = dist + d * d
    d = nx[:, 2:3] - zt_ref[0]
    dist = dist + d * d

    w1 = w1_ref[...]
    xyz = xyz_ref[0]
    x0 = xyz.astype(jnp.bfloat16)
    r = xyz - x0.astype(jnp.float32)
    x1 = r.astype(jnp.bfloat16)
    x2 = (r - x1.astype(jnp.float32)).astype(jnp.bfloat16)
    if has_feat:
        f_hi, f_lo = _split_limbs(feat_ref[0])
    b1 = b1_ref[...]
    w2 = w2_ref[...]
    b2 = b2_ref[...]
    w3 = w3_ref[...]
    b3 = b3_ref[...]

    iota = jax.lax.broadcasted_iota(jnp.int32, (s_blk, n), 1)
    c3 = out_ref.shape[-1]

    def slot(t, carry):
        dist, acc = carry
        m = jnp.min(dist, axis=1, keepdims=True)
        j = jnp.min(jnp.where(dist == m, iota, n), axis=1, keepdims=True)
        onehot = iota == j
        o = onehot.astype(jnp.bfloat16)
        gx = (jnp.dot(o, x0, preferred_element_type=jnp.float32)
              + jnp.dot(o, x1, preferred_element_type=jnp.float32)
              + jnp.dot(o, x2, preferred_element_type=jnp.float32))
        rel = gx - nx
        h = jnp.dot(rel, w1[0:3, :], preferred_element_type=jnp.float32,
                    precision=jax.lax.Precision.HIGHEST)
        if has_feat:
            gf = _onehot_gather(onehot, f_hi, f_lo)
            h = h + jnp.dot(gf, w1[3:, :],
                            preferred_element_type=jnp.float32,
                            precision=jax.lax.Precision.HIGHEST)
        h = jnp.maximum(h + b1, 0.0)
        h = jnp.maximum(
            jnp.dot(h, w2, preferred_element_type=jnp.float32, precision=jax.lax.Precision.HIGHEST) + b2, 0.0)
        h = jnp.maximum(
            jnp.dot(h, w3, preferred_element_type=jnp.float32, precision=jax.lax.Precision.HIGHEST) + b3, 0.0)
        acc = jnp.maximum(acc, h)
        dist = jnp.where(onehot, jnp.float32(jnp.inf), dist)
        return dist, acc

    acc0 = jnp.zeros((s_blk, c3), jnp.float32)
    _, acc = jax.lax.fori_loop(0, _K, slot, (dist, acc0))
    out_ref[0] = acc


def _sa(newxyz, xyz, xt, yt, zt, feat, layers, s_blk):
    bsz, s, _ = newxyz.shape
    n = xyz.shape[1]
    folded = [_fold_bn(l) for l in layers]
    c3 = folded[2][0].shape[1]
    in_specs = [
        pl.BlockSpec((1, s_blk, 3), lambda b, q: (b, q, 0)),
        pl.BlockSpec((1, 1, n), lambda b, q: (b, 0, 0)),
        pl.BlockSpec((1, 1, n), lambda b, q: (b, 0, 0)),
        pl.BlockSpec((1, 1, n), lambda b, q: (b, 0, 0)),
        pl.BlockSpec((1, n, 3), lambda b, q: (b, 0, 0)),
    ]
    args = [newxyz, xt.reshape(bsz, 1, n), yt.reshape(bsz, 1, n),
            zt.reshape(bsz, 1, n), xyz]
    if feat is not None:
        in_specs.append(
            pl.BlockSpec((1, n, feat.shape[-1]), lambda b, q: (b, 0, 0)))
        args.append(feat)
    for w, bb in folded:
        in_specs.append(pl.BlockSpec(w.shape, lambda b, q: (0, 0)))
        in_specs.append(pl.BlockSpec(bb.shape, lambda b, q: (0, 0)))
        args.extend([w, bb])
    return pl.pallas_call(
        functools.partial(_sa_kernel, has_feat=feat is not None, n=n,
                          s_blk=s_blk),
        grid=(bsz, s // s_blk),
        in_specs=in_specs,
        out_specs=pl.BlockSpec((1, s_blk, c3), lambda b, q: (b, q, 0)),
        out_shape=jax.ShapeDtypeStruct((bsz, s, c3), jnp.float32),
        compiler_params=_PAR2,
    )(*args)


def _fp_kernel(*refs, has_feat1, has_head, nlayers, n2, blk, c2):
    q_ref, x2_ref, y2_ref, z2_ref, f2_ref = refs[:5]
    p = 5
    f1_ref = None
    if has_feat1:
        f1_ref = refs[p]
        p += 1
    wb = refs[p:p + 2 * nlayers]
    p += 2 * nlayers
    if has_head:
        hw1_ref, hb1_ref, hw2_ref, hb2_ref = refs[p:p + 4]
        p += 4
    out_ref = refs[p]

    nx = q_ref[0]
    d = nx[:, 0:1] - x2_ref[0]
    dist = d * d
    d = nx[:, 1:2] - y2_ref[0]
    dist = dist + d * d
    d = nx[:, 2:3] - z2_ref[0]
    dist = dist + d * d

    iota = jax.lax.broadcasted_iota(jnp.int32, (blk, n2), 1)
    f2_hi, f2_lo = _split_limbs(f2_ref[0])
    fs = []
    ds = []
    for t in range(3):
        m = jnp.min(dist, axis=1, keepdims=True)
        j = jnp.min(jnp.where(dist == m, iota, n2), axis=1, keepdims=True)
        onehot = iota == j
        ds.append(m)
        fs.append(_onehot_gather(onehot, f2_hi, f2_lo))
        if t < 2:
            dist = jnp.where(onehot, jnp.float32(jnp.inf), dist)

    w0 = 1.0 / jnp.maximum(ds[0], 1e-10)
    w1 = 1.0 / jnp.maximum(ds[1], 1e-10)
    w2 = 1.0 / jnp.maximum(ds[2], 1e-10)
    wsum = w0 + w1 + w2
    interp = fs[0] * (w0 / wsum) + fs[1] * (w1 / wsum)
    interp = interp + fs[2] * (w2 / wsum)

    wl, bl = wb[0], wb[1]
    h = jnp.dot(interp, wl[...][:c2, :], preferred_element_type=jnp.float32, precision=jax.lax.Precision.HIGHEST)
    if has_feat1:
        h = h + jnp.dot(f1_ref[0], wl[...][c2:, :],
                        preferred_element_type=jnp.float32, precision=jax.lax.Precision.HIGHEST)
    h = jnp.maximum(h + bl[...], 0.0)
    for li in range(1, nlayers):
        wl, bl = wb[2 * li], wb[2 * li + 1]
        h = jnp.maximum(
            jnp.dot(h, wl[...], preferred_element_type=jnp.float32, precision=jax.lax.Precision.HIGHEST)
            + bl[...], 0.0)
    if has_head:
        h = jnp.maximum(
            jnp.dot(h, hw1_ref[...], preferred_element_type=jnp.float32, precision=jax.lax.Precision.HIGHEST)
            + hb1_ref[...], 0.0)
        h = jnp.dot(h, hw2_ref[...], preferred_element_type=jnp.float32, precision=jax.lax.Precision.HIGHEST) \
            + hb2_ref[...]
    out_ref[0] = h


def _fp(xyz1, x2t, y2t, z2t, feat1, feat2, layers, blk, head=None):
    bsz, n1, _ = xyz1.shape
    n2 = x2t.shape[1]
    c2 = feat2.shape[-1]
    folded = [_fold_bn(l) for l in layers]
    nlayers = len(folded)
    c_out = folded[-1][0].shape[1]
    in_specs = [
        pl.BlockSpec((1, blk, 3), lambda b, q: (b, q, 0)),
        pl.BlockSpec((1, 1, n2), lambda b, q: (b, 0, 0)),
        pl.BlockSpec((1, 1, n2), lambda b, q: (b, 0, 0)),
        pl.BlockSpec((1, 1, n2), lambda b, q: (b, 0, 0)),
        pl.BlockSpec((1, n2, c2), lambda b, q: (b, 0, 0)),
    ]
    args = [xyz1, x2t.reshape(bsz, 1, n2), y2t.reshape(bsz, 1, n2),
            z2t.reshape(bsz, 1, n2), feat2]
    if feat1 is not None:
        in_specs.append(
            pl.BlockSpec((1, blk, feat1.shape[-1]), lambda b, q: (b, q, 0)))
        args.append(feat1)
    for w, bb in folded:
        in_specs.append(pl.BlockSpec(w.shape, lambda b, q: (0, 0)))
        in_specs.append(pl.BlockSpec(bb.shape, lambda b, q: (0, 0)))
        args.extend([w, bb])
    if head is not None:
        for arr in head:
            in_specs.append(pl.BlockSpec(arr.shape, lambda b, q: (0, 0)))
            args.append(arr)
        c_out = head[2].shape[1]
    return pl.pallas_call(
        functools.partial(_fp_kernel, has_feat1=feat1 is not None,
                          has_head=head is not None, nlayers=nlayers,
                          n2=n2, blk=blk, c2=c2),
        grid=(bsz, n1 // blk),
        in_specs=in_specs,
        out_specs=pl.BlockSpec((1, blk, c_out), lambda b, q: (b, q, 0)),
        out_shape=jax.ShapeDtypeStruct((bsz, n1, c_out), jnp.float32),
        compiler_params=_PAR2,
    )(*args)


def _coords(xyz):
    return xyz[:, :, 0], xyz[:, :, 1], xyz[:, :, 2]


def kernel(x, params):
    x = x.astype(jnp.float32)
    l0 = _coords(x)

    l1_xyz, *l1 = _fps(*l0, 1024)
    l1_p = _sa(l1_xyz, x, *l0, None, params['sa1'], s_blk=512)

    l2_xyz, *l2 = _fps(*l1, 256)
    l2_p = _sa(l2_xyz, l1_xyz, *l1, l1_p, params['sa2'], s_blk=256)

    l3_xyz, *l3 = _fps(*l2, 64)
    l3_p = _sa(l3_xyz, l2_xyz, *l2, l2_p, params['sa3'], s_blk=64)

    l4_xyz, *l4 = _fps(*l3, 16)
    l4_p = _sa(l4_xyz, l3_xyz, *l3, l3_p, params['sa4'], s_blk=16)

    l3_p = _fp(l3_xyz, *l4, l3_p, l4_p, params['fp4'], blk=64)
    l2_p = _fp(l2_xyz, *l3, l2_p, l3_p, params['fp3'], blk=256)
    l1_p = _fp(l1_xyz, *l2, l1_p, l2_p, params['fp2'], blk=256)

    head = params['head']
    hw2 = jnp.zeros((128, 128), jnp.float32).at[:, :13].set(head['W2'])
    hb2 = jnp.zeros((1, 128), jnp.float32).at[:, :13].set(head['b2'])
    head_args = (head['W1'], head['b1'].reshape(1, -1), hw2, hb2)
    logits = _fp(x, *l1, None, l1_p, params['fp1'], blk=512, head=head_args)
    return logits[:, :, :13]


# speedup vs baseline: 1.2167x; 1.2167x over previous
import functools

import jax
import jax.numpy as jnp
from jax.experimental import pallas as pl
from jax.experimental.pallas import tpu as pltpu

_PAR2 = pltpu.CompilerParams(dimension_semantics=("parallel", "parallel"))

_K = 32


def _split_limbs(a):
    a_hi = a.astype(jnp.bfloat16).astype(jnp.float32)
    return a_hi, a - a_hi


def _onehot_gather(onehot, a_hi, a_lo):
    o = onehot.astype(jnp.float32)
    return (jnp.dot(o, a_hi, preferred_element_type=jnp.float32)
            + jnp.dot(o, a_lo, preferred_element_type=jnp.float32))


def _fold_bn(layer):
    scale = layer['gamma'] / jnp.sqrt(layer['var'] + 1e-3)
    w = layer['W'] * scale[None, :]
    b = layer['beta'] - layer['mean'] * scale
    return w, b.reshape(1, -1)


def _fps_kernel(xt_ref, yt_ref, zt_ref, ox_ref, oy_ref, oz_ref, *, npoint, n):
    xs = xt_ref[...]
    ys = yt_ref[...]
    zs = zt_ref[...]
    bsz = xs.shape[0]
    iota = jax.lax.broadcasted_iota(jnp.int32, (bsz, n), 1)
    oiota = jax.lax.broadcasted_iota(jnp.int32, (bsz, npoint), 1)

    def body(i, carry):
        distance, farthest, ax, ay, az = carry
        onehot = iota == farthest
        cx = jnp.sum(jnp.where(onehot, xs, 0.0), axis=1, keepdims=True)
        cy = jnp.sum(jnp.where(onehot, ys, 0.0), axis=1, keepdims=True)
        cz = jnp.sum(jnp.where(onehot, zs, 0.0), axis=1, keepdims=True)
        osel = oiota == i
        ax = ax + jnp.where(osel, cx, 0.0)
        ay = ay + jnp.where(osel, cy, 0.0)
        az = az + jnp.where(osel, cz, 0.0)
        dx = xs - cx
        dy = ys - cy
        dz = zs - cz
        d = dx * dx + dy * dy
        d = d + dz * dz
        distance = jnp.minimum(distance, d)
        m = jnp.max(distance, axis=1, keepdims=True)
        farthest = jnp.min(jnp.where(distance == m, iota, n), axis=1,
                           keepdims=True)
        return distance, farthest, ax, ay, az

    dist0 = jnp.full((bsz, n), 1e10, jnp.float32)
    far0 = jnp.zeros((bsz, 1), jnp.int32)
    acc0 = jnp.zeros((bsz, npoint), jnp.float32)
    _, _, ax, ay, az = jax.lax.fori_loop(
        0, npoint, body, (dist0, far0, acc0, acc0, acc0))
    ox_ref[...] = ax
    oy_ref[...] = ay
    oz_ref[...] = az


def _fps(xt, yt, zt, npoint):
    bsz, n = xt.shape
    shp = jax.ShapeDtypeStruct((bsz, npoint), jnp.float32)
    ox, oy, oz = pl.pallas_call(
        functools.partial(_fps_kernel, npoint=npoint, n=n),
        out_shape=(shp, shp, shp),
    )(xt, yt, zt)
    return jnp.stack([ox, oy, oz], axis=-1), ox, oy, oz


def _sa_kernel(*refs, has_feat, n, s_blk):
    nx_ref, xt_ref, yt_ref, zt_ref, xyz_ref = refs[:5]
    p = 5
    feat_ref = None
    if has_feat:
        feat_ref = refs[p]
        p += 1
    w1_ref, b1_ref, w2_ref, b2_ref, w3_ref, b3_ref = refs[p:p + 6]
    out_ref = refs[p + 6]

    nx = nx_ref[0]
    d = nx[:, 0:1] - xt_ref[0]
    dist = d * d
    d = nx[:, 1:2] - yt_ref[0]
    dist = dist + d * d
    d = nx[:, 2:3] - zt_ref[0]
    dist = dist + d * d

    w1 = w1_ref[...]
    xyz = xyz_ref[0]
    if has_feat:
        a = (jnp.dot(xyz, w1[0:3, :], preferred_element_type=jnp.float32, precision=jax.lax.Precision.HIGHEST)
             + jnp.dot(feat_ref[0], w1[3:, :],
                       preferred_element_type=jnp.float32, precision=jax.lax.Precision.HIGHEST))
    else:
        a = jnp.dot(xyz, w1, preferred_element_type=jnp.float32, precision=jax.lax.Precision.HIGHEST)
    a_hi, a_lo = _split_limbs(a)
    bq = jnp.dot(nx, w1[0:3, :], preferred_element_type=jnp.float32, precision=jax.lax.Precision.HIGHEST)
    b1 = b1_ref[...]
    w2 = w2_ref[...]
    b2 = b2_ref[...]
    w3 = w3_ref[...]
    b3 = b3_ref[...]

    iota = jax.lax.broadcasted_iota(jnp.int32, (s_blk, n), 1)
    c3 = out_ref.shape[-1]

    def slot(t, carry):
        dist, acc = carry
        m = jnp.min(dist, axis=1, keepdims=True)
        j = jnp.min(jnp.where(dist == m, iota, n), axis=1, keepdims=True)
        onehot = iota == j
        asel = _onehot_gather(onehot, a_hi, a_lo)
        h = jnp.maximum(asel - bq + b1, 0.0)
        h = jnp.maximum(
            jnp.dot(h, w2, preferred_element_type=jnp.float32, precision=jax.lax.Precision.HIGHEST) + b2, 0.0)
        h = jnp.maximum(
            jnp.dot(h, w3, preferred_element_type=jnp.float32, precision=jax.lax.Precision.HIGHEST) + b3, 0.0)
        acc = jnp.maximum(acc, h)
        dist = jnp.where(onehot, jnp.float32(jnp.inf), dist)
        return dist, acc

    acc0 = jnp.zeros((s_blk, c3), jnp.float32)
    _, acc = jax.lax.fori_loop(0, _K, slot, (dist, acc0))
    out_ref[0] = acc


def _sa(newxyz, xyz, xt, yt, zt, feat, layers, s_blk):
    bsz, s, _ = newxyz.shape
    n = xyz.shape[1]
    folded = [_fold_bn(l) for l in layers]
    c3 = folded[2][0].shape[1]
    in_specs = [
        pl.BlockSpec((1, s_blk, 3), lambda b, q: (b, q, 0)),
        pl.BlockSpec((1, 1, n), lambda b, q: (b, 0, 0)),
        pl.BlockSpec((1, 1, n), lambda b, q: (b, 0, 0)),
        pl.BlockSpec((1, 1, n), lambda b, q: (b, 0, 0)),
        pl.BlockSpec((1, n, 3), lambda b, q: (b, 0, 0)),
    ]
    args = [newxyz, xt.reshape(bsz, 1, n), yt.reshape(bsz, 1, n),
            zt.reshape(bsz, 1, n), xyz]
    if feat is not None:
        in_specs.append(
            pl.BlockSpec((1, n, feat.shape[-1]), lambda b, q: (b, 0, 0)))
        args.append(feat)
    for w, bb in folded:
        in_specs.append(pl.BlockSpec(w.shape, lambda b, q: (0, 0)))
        in_specs.append(pl.BlockSpec(bb.shape, lambda b, q: (0, 0)))
        args.extend([w, bb])
    return pl.pallas_call(
        functools.partial(_sa_kernel, has_feat=feat is not None, n=n,
                          s_blk=s_blk),
        grid=(bsz, s // s_blk),
        in_specs=in_specs,
        out_specs=pl.BlockSpec((1, s_blk, c3), lambda b, q: (b, q, 0)),
        out_shape=jax.ShapeDtypeStruct((bsz, s, c3), jnp.float32),
        compiler_params=_PAR2,
    )(*args)


def _fp_kernel(*refs, has_feat1, has_head, nlayers, n2, blk, c2):
    q_ref, x2_ref, y2_ref, z2_ref, f2_ref = refs[:5]
    p = 5
    f1_ref = None
    if has_feat1:
        f1_ref = refs[p]
        p += 1
    wb = refs[p:p + 2 * nlayers]
    p += 2 * nlayers
    if has_head:
        hw1_ref, hb1_ref, hw2_ref, hb2_ref = refs[p:p + 4]
        p += 4
    out_ref = refs[p]

    nx = q_ref[0]
    d = nx[:, 0:1] - x2_ref[0]
    dist = d * d
    d = nx[:, 1:2] - y2_ref[0]
    dist = dist + d * d
    d = nx[:, 2:3] - z2_ref[0]
    dist = dist + d * d

    iota = jax.lax.broadcasted_iota(jnp.int32, (blk, n2), 1)
    f2_hi, f2_lo = _split_limbs(f2_ref[0])
    fs = []
    ds = []
    for t in range(3):
        m = jnp.min(dist, axis=1, keepdims=True)
        j = jnp.min(jnp.where(dist == m, iota, n2), axis=1, keepdims=True)
        onehot = iota == j
        ds.append(m)
        fs.append(_onehot_gather(onehot, f2_hi, f2_lo))
        if t < 2:
            dist = jnp.where(onehot, jnp.float32(jnp.inf), dist)

    w0 = 1.0 / jnp.maximum(ds[0], 1e-10)
    w1 = 1.0 / jnp.maximum(ds[1], 1e-10)
    w2 = 1.0 / jnp.maximum(ds[2], 1e-10)
    wsum = w0 + w1 + w2
    interp = fs[0] * (w0 / wsum) + fs[1] * (w1 / wsum)
    interp = interp + fs[2] * (w2 / wsum)

    wl, bl = wb[0], wb[1]
    h = jnp.dot(interp, wl[...][:c2, :], preferred_element_type=jnp.float32, precision=jax.lax.Precision.HIGHEST)
    if has_feat1:
        h = h + jnp.dot(f1_ref[0], wl[...][c2:, :],
                        preferred_element_type=jnp.float32, precision=jax.lax.Precision.HIGHEST)
    h = jnp.maximum(h + bl[...], 0.0)
    for li in range(1, nlayers):
        wl, bl = wb[2 * li], wb[2 * li + 1]
        h = jnp.maximum(
            jnp.dot(h, wl[...], preferred_element_type=jnp.float32, precision=jax.lax.Precision.HIGHEST)
            + bl[...], 0.0)
    if has_head:
        h = jnp.maximum(
            jnp.dot(h, hw1_ref[...], preferred_element_type=jnp.float32, precision=jax.lax.Precision.HIGHEST)
            + hb1_ref[...], 0.0)
        h = jnp.dot(h, hw2_ref[...], preferred_element_type=jnp.float32, precision=jax.lax.Precision.HIGHEST) \
            + hb2_ref[...]
    out_ref[0] = h


def _fp(xyz1, x2t, y2t, z2t, feat1, feat2, layers, blk, head=None):
    bsz, n1, _ = xyz1.shape
    n2 = x2t.shape[1]
    c2 = feat2.shape[-1]
    folded = [_fold_bn(l) for l in layers]
    nlayers = len(folded)
    c_out = folded[-1][0].shape[1]
    in_specs = [
        pl.BlockSpec((1, blk, 3), lambda b, q: (b, q, 0)),
        pl.BlockSpec((1, 1, n2), lambda b, q: (b, 0, 0)),
        pl.BlockSpec((1, 1, n2), lambda b, q: (b, 0, 0)),
        pl.BlockSpec((1, 1, n2), lambda b, q: (b, 0, 0)),
        pl.BlockSpec((1, n2, c2), lambda b, q: (b, 0, 0)),
    ]
    args = [xyz1, x2t.reshape(bsz, 1, n2), y2t.reshape(bsz, 1, n2),
            z2t.reshape(bsz, 1, n2), feat2]
    if feat1 is not None:
        in_specs.append(
            pl.BlockSpec((1, blk, feat1.shape[-1]), lambda b, q: (b, q, 0)))
        args.append(feat1)
    for w, bb in folded:
        in_specs.append(pl.BlockSpec(w.shape, lambda b, q: (0, 0)))
        in_specs.append(pl.BlockSpec(bb.shape, lambda b, q: (0, 0)))
        args.extend([w, bb])
    if head is not None:
        for arr in head:
            in_specs.append(pl.BlockSpec(arr.shape, lambda b, q: (0, 0)))
            args.append(arr)
        c_out = head[2].shape[1]
    return pl.pallas_call(
        functools.partial(_fp_kernel, has_feat1=feat1 is not None,
                          has_head=head is not None, nlayers=nlayers,
                          n2=n2, blk=blk, c2=c2),
        grid=(bsz, n1 // blk),
        in_specs=in_specs,
        out_specs=pl.BlockSpec((1, blk, c_out), lambda b, q: (b, q, 0)),
        out_shape=jax.ShapeDtypeStruct((bsz, n1, c_out), jnp.float32),
        compiler_params=_PAR2,
    )(*args)


def _coords(xyz):
    return xyz[:, :, 0], xyz[:, :, 1], xyz[:, :, 2]


def kernel(x, params):
    x = x.astype(jnp.float32)
    l0 = _coords(x)

    l1_xyz, *l1 = _fps(*l0, 1024)
    l1_p = _sa(l1_xyz, x, *l0, None, params['sa1'], s_blk=512)

    l2_xyz, *l2 = _fps(*l1, 256)
    l2_p = _sa(l2_xyz, l1_xyz, *l1, l1_p, params['sa2'], s_blk=256)

    l3_xyz, *l3 = _fps(*l2, 64)
    l3_p = _sa(l3_xyz, l2_xyz, *l2, l2_p, params['sa3'], s_blk=64)

    l4_xyz, *l4 = _fps(*l3, 16)
    l4_p = _sa(l4_xyz, l3_xyz, *l3, l3_p, params['sa4'], s_blk=16)

    l3_p = _fp(l3_xyz, *l4, l3_p, l4_p, params['fp4'], blk=64)
    l2_p = _fp(l2_xyz, *l3, l2_p, l3_p, params['fp3'], blk=256)
    l1_p = _fp(l1_xyz, *l2, l1_p, l2_p, params['fp2'], blk=256)

    head = params['head']
    hw2 = jnp.zeros((128, 128), jnp.float32).at[:, :13].set(head['W2'])
    hb2 = jnp.zeros((1, 128), jnp.float32).at[:, :13].set(head['b2'])
    head_args = (head['W1'], head['b1'].reshape(1, -1), hw2, hb2)
    logits = _fp(x, *l1, None, l1_p, params['fp1'], blk=512, head=head_args)
    return logits[:, :, :13]
